# baseline (speedup 1.0000x reference)
import functools

import jax
import jax.numpy as jnp
from jax import lax
from jax.experimental import pallas as pl
from jax.experimental.pallas import tpu as pltpu
from jax.experimental.pallas import tpu_sc as plsc

B, L, F = 4096, 50, 10
D = 32
NT = 8
G = 128
V = 16
DT = D // 8
BT = B // G

NC, NS = 2, 16
NW = NC * NS

NG = NT + 1

_MESH = plsc.VectorSubcoreMesh(core_axis_name="c", subcore_axis_name="s")


@functools.partial(
    pl.kernel,
    out_type=(
        jax.ShapeDtypeStruct((L, DT, BT, 8, G), jnp.float32),
        jax.ShapeDtypeStruct((L, NT, DT, BT, 8, G), jnp.float32),
    ),
    mesh=_MESH,
    compiler_params=pltpu.CompilerParams(
        use_tc_tiling_on_sc=False, needs_layout_passes=False),
    scratch_types=(
        pltpu.VMEM((2, F, G), jnp.int32),
        pltpu.VMEM((2, NG, G, D), jnp.float32),
        pltpu.VMEM((NG, D, G), jnp.float32),
        pltpu.SemaphoreType.DMA,
        pltpu.SemaphoreType.DMA,
        pltpu.SemaphoreType.DMA,
    ),
)
def _sc_embed(xt_hbm, rel_tab_hbm, type_tab_hbm, rel_out_hbm, type_out_hbm,
              idx_v, rows_v, tbuf_v, sem0, sem1, semo):
    wid = lax.axis_index("s") * NC + lax.axis_index("c")
    sems = (sem0, sem1)
    bt = wid

    lane = lax.iota(jnp.int32, V)

    def load_fire(slot, l):
        pltpu.sync_copy(xt_hbm.at[:, l, pl.ds(bt * G, G)], idx_v.at[slot])
        for j in range(NT):
            pltpu.async_copy(
                type_tab_hbm.at[idx_v.at[slot].at[j]],
                rows_v.at[slot].at[j], sems[slot])
        pltpu.async_copy(
            rel_tab_hbm.at[idx_v.at[slot].at[F - 1]],
            rows_v.at[slot].at[NT], sems[slot])

    def drain_streams(slot):
        for j in range(NG):
            pltpu.make_async_copy(
                type_tab_hbm.at[idx_v.at[slot].at[0]],
                rows_v.at[slot].at[j], sems[slot]).wait()

    def drain_stores(l):
        for j in range(NT):
            for dt in range(DT):
                pltpu.make_async_copy(
                    tbuf_v.at[j].at[pl.ds(dt * 8, 8)],
                    type_out_hbm.at[l, j, dt, bt], semo).wait()
        for dt in range(DT):
            pltpu.make_async_copy(
                tbuf_v.at[NT].at[pl.ds(dt * 8, 8)],
                rel_out_hbm.at[l, dt, bt], semo).wait()

    def transpose_store(slot, l):
        for j in range(NG):
            rows = rows_v.at[slot].at[j]
            trow = tbuf_v.at[j]

            def dstep(i, carry, rows=rows, trow=trow):
                for u in range(4):
                    d = i * 4 + u
                    col = jnp.full((V,), d, jnp.int32)
                    for c in range(0, G, V):
                        trow[d, pl.ds(c, V)] = plsc.load_gather(
                            rows, [lane + c, col])
                return carry

            lax.fori_loop(0, D // 4, dstep, 0)
        for j in range(NT):
            for dt in range(DT):
                pltpu.async_copy(
                    tbuf_v.at[j].at[pl.ds(dt * 8, 8)],
                    type_out_hbm.at[l, j, dt, bt], semo)
        for dt in range(DT):
            pltpu.async_copy(
                tbuf_v.at[NT].at[pl.ds(dt * 8, 8)],
                rel_out_hbm.at[l, dt, bt], semo)

    n_pairs = L // 2
    load_fire(0, 0)

    def pair(p, carry):
        l0 = 2 * p
        load_fire(1, l0 + 1)
        drain_streams(0)

        @pl.when(p > 0)
        def _():
            drain_stores(l0 - 1)

        transpose_store(0, l0)

        @pl.when(p < n_pairs - 1)
        def _():
            load_fire(0, l0 + 2)

        drain_streams(1)
        drain_stores(l0)
        transpose_store(1, l0 + 1)
        return carry

    lax.fori_loop(0, n_pairs, pair, 0)
    drain_stores(L - 1)


def kernel(x, rel_table, type_table):
    xt = jnp.transpose(x, (2, 1, 0))
    rel6, type6 = _sc_embed(xt, rel_table, type_table)
    rel = rel6.transpose(2, 4, 0, 1, 3).reshape(B, L, D)
    typ = type6.transpose(3, 5, 0, 1, 2, 4).reshape(B, L, NT, D)
    return (rel, typ)


# speedup vs baseline: 1.7007x; 1.7007x over previous
import functools

import jax
import jax.numpy as jnp
from jax import lax
from jax.experimental import pallas as pl
from jax.experimental.pallas import tpu as pltpu
from jax.experimental.pallas import tpu_sc as plsc

B, L, F = 4096, 50, 10
D = 32
NT = 8
G = 128
V = 16
DT = D // 8
BT = B // G

NC, NS = 2, 16
NW = NC * NS

NG = NT + 1

_MESH = plsc.VectorSubcoreMesh(core_axis_name="c", subcore_axis_name="s")


@functools.partial(
    pl.kernel,
    out_type=(
        jax.ShapeDtypeStruct((L, DT, BT, 8, G), jnp.float32),
        jax.ShapeDtypeStruct((L, NT, DT, BT, 8, G), jnp.float32),
    ),
    mesh=_MESH,
    compiler_params=pltpu.CompilerParams(
        use_tc_tiling_on_sc=False, needs_layout_passes=False),
    scratch_types=(
        pltpu.VMEM((2, F, G), jnp.int32),
        pltpu.VMEM((2, NG, G, D), jnp.float32),
        pltpu.VMEM((NG, D, G), jnp.float32),
        pltpu.SemaphoreType.DMA,
        pltpu.SemaphoreType.DMA,
        pltpu.SemaphoreType.DMA,
    ),
)
def _sc_embed(xt_hbm, rel_tab_hbm, type_tab_hbm, rel_out_hbm, type_out_hbm,
              idx_v, rows_v, tbuf_v, sem0, sem1, semo):
    wid = lax.axis_index("s") * NC + lax.axis_index("c")
    sems = (sem0, sem1)
    bt = wid

    lane = lax.iota(jnp.int32, V)

    def load_fire(slot, l):
        pltpu.sync_copy(xt_hbm.at[:, l, pl.ds(bt * G, G)], idx_v.at[slot])
        for j in range(NT):
            pltpu.async_copy(
                type_tab_hbm.at[idx_v.at[slot].at[j]],
                rows_v.at[slot].at[j], sems[slot])
        pltpu.async_copy(
            rel_tab_hbm.at[idx_v.at[slot].at[F - 1]],
            rows_v.at[slot].at[NT], sems[slot])

    def drain_streams(slot):
        for j in range(NG):
            pltpu.make_async_copy(
                type_tab_hbm.at[idx_v.at[slot].at[0]],
                rows_v.at[slot].at[j], sems[slot]).wait()

    def drain_stores(l):
        for j in range(NT):
            for dt in range(DT):
                pltpu.make_async_copy(
                    tbuf_v.at[j].at[pl.ds(dt * 8, 8)],
                    type_out_hbm.at[l, j, dt, bt], semo).wait()
        for dt in range(DT):
            pltpu.make_async_copy(
                tbuf_v.at[NT].at[pl.ds(dt * 8, 8)],
                rel_out_hbm.at[l, dt, bt], semo).wait()

    def transpose_store(slot, l):
        for j in range(NG):
            rows = rows_v.at[slot].at[j]
            trow = tbuf_v.at[j]

            @plsc.parallel_loop(0, D, step=1, unroll=4)
            def _t(d, rows=rows, trow=trow):
                col = jnp.full((V,), d, jnp.int32)
                for c in range(0, G, V):
                    trow[d, pl.ds(c, V)] = plsc.load_gather(
                        rows, [lane + c, col])
        for j in range(NT):
            for dt in range(DT):
                pltpu.async_copy(
                    tbuf_v.at[j].at[pl.ds(dt * 8, 8)],
                    type_out_hbm.at[l, j, dt, bt], semo)
        for dt in range(DT):
            pltpu.async_copy(
                tbuf_v.at[NT].at[pl.ds(dt * 8, 8)],
                rel_out_hbm.at[l, dt, bt], semo)

    n_pairs = L // 2
    load_fire(0, 0)

    def pair(p, carry):
        l0 = 2 * p
        load_fire(1, l0 + 1)
        drain_streams(0)

        @pl.when(p > 0)
        def _():
            drain_stores(l0 - 1)

        transpose_store(0, l0)

        @pl.when(p < n_pairs - 1)
        def _():
            load_fire(0, l0 + 2)

        drain_streams(1)
        drain_stores(l0)
        transpose_store(1, l0 + 1)
        return carry

    lax.fori_loop(0, n_pairs, pair, 0)
    drain_stores(L - 1)


def kernel(x, rel_table, type_table):
    xt = jnp.transpose(x, (2, 1, 0))
    rel6, type6 = _sc_embed(xt, rel_table, type_table)
    rel = rel6.transpose(2, 4, 0, 1, 3).reshape(B, L, D)
    typ = type6.transpose(3, 5, 0, 1, 2, 4).reshape(B, L, NT, D)
    return (rel, typ)


# speedup vs baseline: 1.7283x; 1.0163x over previous
import functools

import jax
import jax.numpy as jnp
from jax import lax
from jax.experimental import pallas as pl
from jax.experimental.pallas import tpu as pltpu
from jax.experimental.pallas import tpu_sc as plsc

B, L, F = 4096, 50, 10
D = 32
NT = 8
G = 128
V = 16
DT = D // 8
BT = B // G

NC, NS = 2, 16
NW = NC * NS

NG = NT + 1

_MESH = plsc.VectorSubcoreMesh(core_axis_name="c", subcore_axis_name="s")


@functools.partial(
    pl.kernel,
    out_type=(
        jax.ShapeDtypeStruct((L, DT, BT, 8, G), jnp.float32),
        jax.ShapeDtypeStruct((L, NT, DT, BT, 8, G), jnp.float32),
    ),
    mesh=_MESH,
    compiler_params=pltpu.CompilerParams(
        use_tc_tiling_on_sc=False, needs_layout_passes=False),
    scratch_types=(
        pltpu.VMEM((2, F, G), jnp.int32),
        pltpu.VMEM((2, NG, G, D), jnp.float32),
        pltpu.VMEM((NG, D, G), jnp.float32),
        pltpu.SemaphoreType.DMA,
        pltpu.SemaphoreType.DMA,
        pltpu.SemaphoreType.DMA,
        pltpu.SemaphoreType.DMA,
        pltpu.SemaphoreType.DMA,
    ),
)
def _sc_embed(xt_hbm, rel_tab_hbm, type_tab_hbm, rel_out_hbm, type_out_hbm,
              idx_v, rows_v, tbuf_v, sem0, sem1, semi0, semi1, semo):
    wid = lax.axis_index("s") * NC + lax.axis_index("c")
    sems = (sem0, sem1)
    semis = (semi0, semi1)
    bt = wid

    lane = lax.iota(jnp.int32, V)

    def start_idx(slot, l):
        pltpu.async_copy(xt_hbm.at[:, l, pl.ds(bt * G, G)], idx_v.at[slot],
                         semis[slot])

    def fire_streams(slot, l):
        pltpu.make_async_copy(xt_hbm.at[:, l, pl.ds(bt * G, G)],
                              idx_v.at[slot], semis[slot]).wait()
        for j in range(NT):
            pltpu.async_copy(
                type_tab_hbm.at[idx_v.at[slot].at[j]],
                rows_v.at[slot].at[j], sems[slot])
        pltpu.async_copy(
            rel_tab_hbm.at[idx_v.at[slot].at[F - 1]],
            rows_v.at[slot].at[NT], sems[slot])

    def drain_streams(slot):
        for j in range(NG):
            pltpu.make_async_copy(
                type_tab_hbm.at[idx_v.at[slot].at[0]],
                rows_v.at[slot].at[j], sems[slot]).wait()

    def drain_stores(l):
        for j in range(NT):
            for dt in range(DT):
                pltpu.make_async_copy(
                    tbuf_v.at[j].at[pl.ds(dt * 8, 8)],
                    type_out_hbm.at[l, j, dt, bt], semo).wait()
        for dt in range(DT):
            pltpu.make_async_copy(
                tbuf_v.at[NT].at[pl.ds(dt * 8, 8)],
                rel_out_hbm.at[l, dt, bt], semo).wait()

    def transpose_store(slot, l):
        rows3 = rows_v.at[slot]

        @plsc.parallel_loop(0, NG * D, step=1, unroll=8)
        def _t(r):
            j = r // D
            d = r % D
            jcol = jnp.full((V,), j, jnp.int32)
            dcol = jnp.full((V,), d, jnp.int32)
            for c in range(0, G, V):
                tbuf_v[j, d, pl.ds(c, V)] = plsc.load_gather(
                    rows3, [jcol, lane + c, dcol])

        for j in range(NT):
            for dt in range(DT):
                pltpu.async_copy(
                    tbuf_v.at[j].at[pl.ds(dt * 8, 8)],
                    type_out_hbm.at[l, j, dt, bt], semo)
        for dt in range(DT):
            pltpu.async_copy(
                tbuf_v.at[NT].at[pl.ds(dt * 8, 8)],
                rel_out_hbm.at[l, dt, bt], semo)

    n_pairs = L // 2
    start_idx(0, 0)
    fire_streams(0, 0)

    def pair(p, carry):
        l0 = 2 * p
        start_idx(1, l0 + 1)
        drain_streams(0)

        @pl.when(p > 0)
        def _():
            drain_stores(l0 - 1)

        fire_streams(1, l0 + 1)
        transpose_store(0, l0)

        @pl.when(p < n_pairs - 1)
        def _():
            start_idx(0, l0 + 2)

        drain_streams(1)
        drain_stores(l0)

        @pl.when(p < n_pairs - 1)
        def _():
            fire_streams(0, l0 + 2)

        transpose_store(1, l0 + 1)
        return carry

    lax.fori_loop(0, n_pairs, pair, 0)
    drain_stores(L - 1)


def kernel(x, rel_table, type_table):
    xt = jnp.transpose(x, (2, 1, 0))
    rel6, type6 = _sc_embed(xt, rel_table, type_table)
    rel = rel6.transpose(2, 4, 0, 1, 3).reshape(B, L, D)
    typ = type6.transpose(3, 5, 0, 1, 2, 4).reshape(B, L, NT, D)
    return (rel, typ)
